# Initial kernel scaffold; baseline (speedup 1.0000x reference)
import jax
import jax.numpy as jnp
from jax.experimental import pallas as pl


def kernel(x, edge_index, batch, W_enc, b_enc, W_gat, att_src, att_dst, b_gat, n1_w, n1_b, n1_ms, W_tag, b_tag, n2_w, n2_b, n2_ms, W_rel, b_rel, W_root, W_c1, b_c1, W_c2, b_c2):
    raise NotImplementedError("write your pallas kernel here")


# speedup vs baseline: 1.0007x; 1.0007x over previous
import jax
import jax.numpy as jnp
from jax.experimental import pallas as pl

N = 50000
E = 800000
B = 64
F = 8
H = 64
HEADS = 4
DH = 16
K = 3
NC = 2


def _seg_softmax(scores, seg, n):
    m = jax.ops.segment_max(scores, seg, num_segments=n)
    m = jnp.where(jnp.isfinite(m), m, 0.0)
    e = jnp.exp(scores - m[seg])
    s = jax.ops.segment_sum(e, seg, num_segments=n)
    return e / (s[seg] + 1e-16)


def _gat(x, ei, W, att_s, att_d, b):
    n = x.shape[0]
    sl = jnp.arange(n, dtype=ei.dtype)
    ei = jnp.concatenate([ei, jnp.stack([sl, sl])], axis=1)
    src, dst = ei[0], ei[1]
    h = (x @ W).reshape(n, HEADS, DH)
    a_s = (h * att_s[None]).sum(-1)
    a_d = (h * att_d[None]).sum(-1)
    alpha = jax.nn.leaky_relu(a_s[src] + a_d[dst], negative_slope=0.2)
    alpha = _seg_softmax(alpha, dst, n)
    out = jax.ops.segment_sum(h[src] * alpha[:, :, None], dst, num_segments=n)
    return out.reshape(n, HEADS * DH) + b


def _graph_norm(x, batch, w, bias, ms):
    cnt = jnp.maximum(jax.ops.segment_sum(jnp.ones((x.shape[0],), x.dtype), batch, num_segments=B), 1.0)[:, None]
    mean = jax.ops.segment_sum(x, batch, num_segments=B) / cnt
    out = x - mean[batch] * ms
    var = jax.ops.segment_sum(out * out, batch, num_segments=B) / cnt
    std = jnp.sqrt(var + 1e-5)[batch]
    return w * out / std + bias


def _tag(x, ei, Ws, b):
    n = x.shape[0]
    src, dst = ei[0], ei[1]
    deg = jax.ops.segment_sum(jnp.ones((ei.shape[1],), x.dtype), dst, num_segments=n)
    dis = jnp.where(deg > 0, 1.0 / jnp.sqrt(deg), 0.0)
    norm = (dis[src] * dis[dst])[:, None]
    out = x @ Ws[0]
    h = x
    for k in range(1, K + 1):
        h = jax.ops.segment_sum(h[src] * norm, dst, num_segments=n)
        out = out + h @ Ws[k]
    return out + b


def _gconv(x, ei, Wr, br, Wroot):
    n = x.shape[0]
    agg = jax.ops.segment_sum(x[ei[0]], ei[1], num_segments=n)
    return agg @ Wr + br + x @ Wroot


def _noop_pallas(x):
    def body(x_ref, o_ref):
        o_ref[...] = x_ref[...]
    return pl.pallas_call(body, out_shape=jax.ShapeDtypeStruct(x.shape, x.dtype))(x)


def kernel(x, edge_index, batch, W_enc, b_enc, W_gat, att_src, att_dst, b_gat, n1_w, n1_b, n1_ms, W_tag, b_tag, n2_w, n2_b, n2_ms, W_rel, b_rel, W_root, W_c1, b_c1, W_c2, b_c2):
    x = jax.nn.gelu(x @ W_enc + b_enc, approximate=False)
    x = jax.nn.relu(_graph_norm(_gat(x, edge_index, W_gat, att_src, att_dst, b_gat), batch, n1_w, n1_b, n1_ms))
    x = jax.nn.relu(_graph_norm(_tag(x, edge_index, W_tag, b_tag), batch, n2_w, n2_b, n2_ms))
    x = jax.nn.relu(_gconv(x, edge_index, W_rel, b_rel, W_root))
    cnt = jnp.maximum(jax.ops.segment_sum(jnp.ones((x.shape[0],), x.dtype), batch, num_segments=B), 1.0)[:, None]
    xm = jax.ops.segment_max(x, batch, num_segments=B)
    xm = jnp.where(jnp.isfinite(xm), xm, 0.0)
    xM = jax.ops.segment_sum(x, batch, num_segments=B) / cnt
    z = jnp.concatenate([xm, xM], axis=1)
    z = jax.nn.gelu(z @ W_c1 + b_c1, approximate=False)
    out = z @ W_c2 + b_c2
    out = _noop_pallas(out)
    return jax.nn.log_softmax(out, axis=-1)


# speedup vs baseline: 1.2101x; 1.2093x over previous
import functools

import jax
import jax.numpy as jnp
from jax import lax
from jax.experimental import pallas as pl
from jax.experimental.pallas import tpu as pltpu
from jax.experimental.pallas import tpu_sc as plsc

N = 50000
E = 800000
B = 64
F = 8
H = 64
HEADS = 4
DH = 16
K = 3
NC = 2

NP = 51200
HW = 32
CH = 80
BLK = 2000
NBUF = 5
EPT = E // 16
NBLK = EPT // BLK
NCHB = BLK // CH
ROWS_PT = NP // 16


def _sc_hop_body(h2, src, dst2d, out, acc, zbuf, idxb, dstb, rows, *gsems):
    c = lax.axis_index("c")
    s = lax.axis_index("s")

    def _z(i, _):
        zbuf[i, pl.ds(0, 16)] = jnp.zeros((16,), jnp.float32)
        zbuf[i, pl.ds(16, 16)] = jnp.zeros((16,), jnp.float32)
        return 0
    lax.fori_loop(0, 128, _z, 0)

    def _zcp(i, _):
        pltpu.sync_copy(zbuf, acc.at[pl.ds(s * ROWS_PT + i * 128, 128)])
        return 0
    lax.fori_loop(0, ROWS_PT // 128, _zcp, 0)
    plsc.subcore_barrier()

    ebase = s * EPT
    coff = c * NP

    def _gather(ch_in_blk, b):
        pltpu.async_copy(
            h2.at[idxb.at[pl.ds(ch_in_blk * CH, CH)]], rows.at[b], gsems[b])

    def _block(blk, _):
        pltpu.sync_copy(src.at[pl.ds(ebase + blk * BLK, BLK)], idxb)
        pltpu.sync_copy(dst2d.at[pl.ds(s * (EPT // CH) + blk * NCHB, NCHB)], dstb)

        def _off(i, _):
            idxb[pl.ds(i * 16, 16)] = idxb[pl.ds(i * 16, 16)] + coff
            return 0
        lax.fori_loop(0, BLK // 16, _off, 0)

        for b in range(NBUF):
            _gather(b, b)

        def _grp(g, _):
            k0 = g * NBUF
            for b in range(NBUF):
                ch = k0 + b
                pltpu.make_async_copy(
                    h2.at[pl.ds(0, CH)], rows.at[b], gsems[b]).wait()
                pltpu.sync_copy(rows.at[b], acc.at[dstb.at[ch]], add=True)
                _gather(ch + NBUF, b)
            return 0
        lax.fori_loop(0, (NCHB - NBUF) // NBUF, _grp, 0)

        for b in range(NBUF):
            ch = NCHB - NBUF + b
            pltpu.make_async_copy(
                h2.at[pl.ds(0, CH)], rows.at[b], gsems[b]).wait()
            pltpu.sync_copy(rows.at[b], acc.at[dstb.at[ch]], add=True)
        return 0

    lax.fori_loop(0, NBLK, _block, 0)

    plsc.subcore_barrier()
    pltpu.sync_copy(acc.at[pl.ds(s * ROWS_PT, ROWS_PT)],
                    out.at[pl.ds(coff + s * ROWS_PT, ROWS_PT)])


@jax.jit
def _sc_hop(h2, src, dst2d):
    mesh = plsc.VectorSubcoreMesh(core_axis_name="c", subcore_axis_name="s")
    f = pl.kernel(
        _sc_hop_body,
        out_type=jax.ShapeDtypeStruct((2 * NP, HW), jnp.float32),
        mesh=mesh,
        scratch_types=[
            pltpu.VMEM_SHARED((NP, HW), jnp.float32),
            pltpu.VMEM((128, HW), jnp.float32),
            pltpu.VMEM((BLK,), jnp.int32),
            pltpu.VMEM((NCHB, CH), jnp.int32),
            pltpu.VMEM((NBUF, CH, HW), jnp.float32),
        ] + [pltpu.SemaphoreType.DMA] * NBUF,
        compiler_params=pltpu.CompilerParams(use_tc_tiling_on_sc=False),
    )
    return f(h2, src, dst2d)


def _split_cols(g):
    lo = jnp.pad(g[:, :HW], ((0, NP - N), (0, 0)))
    hi = jnp.pad(g[:, HW:], ((0, NP - N), (0, 0)))
    return jnp.concatenate([lo, hi], axis=0)


def _merge_cols(a2):
    return jnp.concatenate([a2[:N], a2[NP:NP + N]], axis=1)


def _seg_softmax(scores, seg, n):
    m = jax.ops.segment_max(scores, seg, num_segments=n)
    m = jnp.where(jnp.isfinite(m), m, 0.0)
    e = jnp.exp(scores - m[seg])
    s = jax.ops.segment_sum(e, seg, num_segments=n)
    return e / (s[seg] + 1e-16)


def _gat(x, ei, W, att_s, att_d, b):
    n = x.shape[0]
    sl = jnp.arange(n, dtype=ei.dtype)
    ei = jnp.concatenate([ei, jnp.stack([sl, sl])], axis=1)
    src, dst = ei[0], ei[1]
    h = (x @ W).reshape(n, HEADS, DH)
    a_s = (h * att_s[None]).sum(-1)
    a_d = (h * att_d[None]).sum(-1)
    alpha = jax.nn.leaky_relu(a_s[src] + a_d[dst], negative_slope=0.2)
    alpha = _seg_softmax(alpha, dst, n)
    out = jax.ops.segment_sum(h[src] * alpha[:, :, None], dst, num_segments=n)
    return out.reshape(n, HEADS * DH) + b


def _graph_norm(x, batch, w, bias, ms):
    cnt = jnp.maximum(jax.ops.segment_sum(jnp.ones((x.shape[0],), x.dtype), batch, num_segments=B), 1.0)[:, None]
    mean = jax.ops.segment_sum(x, batch, num_segments=B) / cnt
    out = x - mean[batch] * ms
    var = jax.ops.segment_sum(out * out, batch, num_segments=B) / cnt
    std = jnp.sqrt(var + 1e-5)[batch]
    return w * out / std + bias


def kernel(x, edge_index, batch, W_enc, b_enc, W_gat, att_src, att_dst, b_gat, n1_w, n1_b, n1_ms, W_tag, b_tag, n2_w, n2_b, n2_ms, W_rel, b_rel, W_root, W_c1, b_c1, W_c2, b_c2):
    src = edge_index[0]
    dst2d = edge_index[1].reshape(E // CH, CH)

    x1 = jax.nn.gelu(x @ W_enc + b_enc, approximate=False)
    xg = jax.nn.relu(_graph_norm(_gat(x1, edge_index, W_gat, att_src, att_dst, b_gat), batch, n1_w, n1_b, n1_ms))

    deg = jax.ops.segment_sum(jnp.ones((E,), jnp.float32), edge_index[1], num_segments=N)
    dis = jnp.where(deg > 0, 1.0 / jnp.sqrt(deg), 0.0)[:, None]
    out = xg @ W_tag[0]
    h = xg
    for k in range(1, K + 1):
        agg = _merge_cols(_sc_hop(_split_cols(h * dis), src, dst2d))
        h = agg * dis
        out = out + h @ W_tag[k]
    xt = jax.nn.relu(_graph_norm(out + b_tag, batch, n2_w, n2_b, n2_ms))

    agg = _merge_cols(_sc_hop(_split_cols(xt), src, dst2d))
    xc = jax.nn.relu(agg @ W_rel + b_rel + xt @ W_root)

    cnt = jnp.maximum(jax.ops.segment_sum(jnp.ones((N,), jnp.float32), batch, num_segments=B), 1.0)[:, None]
    xm = jax.ops.segment_max(xc, batch, num_segments=B)
    xm = jnp.where(jnp.isfinite(xm), xm, 0.0)
    xM = jax.ops.segment_sum(xc, batch, num_segments=B) / cnt
    z = jnp.concatenate([xm, xM], axis=1)
    z = jax.nn.gelu(z @ W_c1 + b_c1, approximate=False)
    out = z @ W_c2 + b_c2
    return jax.nn.log_softmax(out, axis=-1)


# speedup vs baseline: 28.7446x; 23.7541x over previous
import functools

import jax
import jax.numpy as jnp
from jax import lax
from jax.experimental import pallas as pl
from jax.experimental.pallas import tpu as pltpu
from jax.experimental.pallas import tpu_sc as plsc

N = 50000
E = 800000
B = 64
F = 8
H = 64
HEADS = 4
DH = 16
K = 3
NC = 2

NP = 51200
HW = 32
CH = 80
BLK = 2000
NBUF = 5
EPT = E // 16
NBLK = EPT // BLK
NCHB = BLK // CH
ROWS_PT = NP // 16


def _sc_hop_body(h2, src, dst2d, out, acc, zbuf, idxb, dstb, rows, *gsems):
    c = lax.axis_index("c")
    s = lax.axis_index("s")

    def _z(i, _):
        zbuf[i, pl.ds(0, 16)] = jnp.zeros((16,), jnp.float32)
        zbuf[i, pl.ds(16, 16)] = jnp.zeros((16,), jnp.float32)
        return 0
    lax.fori_loop(0, 128, _z, 0)

    def _zcp(i, _):
        pltpu.sync_copy(zbuf, acc.at[pl.ds(s * ROWS_PT + i * 128, 128)])
        return 0
    lax.fori_loop(0, ROWS_PT // 128, _zcp, 0)
    plsc.subcore_barrier()

    ebase = s * EPT
    coff = c * NP

    def _gather(ch_in_blk, b):
        pltpu.async_copy(
            h2.at[idxb.at[pl.ds(ch_in_blk * CH, CH)]], rows.at[b], gsems[b])

    def _block(blk, _):
        pltpu.sync_copy(src.at[pl.ds(ebase + blk * BLK, BLK)], idxb)
        pltpu.sync_copy(dst2d.at[pl.ds(s * (EPT // CH) + blk * NCHB, NCHB)], dstb)

        def _off(i, _):
            idxb[pl.ds(i * 16, 16)] = idxb[pl.ds(i * 16, 16)] + coff
            return 0
        lax.fori_loop(0, BLK // 16, _off, 0)

        for b in range(NBUF):
            _gather(b, b)

        def _grp(g, _):
            k0 = g * NBUF
            for b in range(NBUF):
                ch = k0 + b
                pltpu.make_async_copy(
                    h2.at[pl.ds(0, CH)], rows.at[b], gsems[b]).wait()
                pltpu.sync_copy(rows.at[b], acc.at[dstb.at[ch]], add=True)
                _gather(ch + NBUF, b)
            return 0
        lax.fori_loop(0, (NCHB - NBUF) // NBUF, _grp, 0)

        for b in range(NBUF):
            ch = NCHB - NBUF + b
            pltpu.make_async_copy(
                h2.at[pl.ds(0, CH)], rows.at[b], gsems[b]).wait()
            pltpu.sync_copy(rows.at[b], acc.at[dstb.at[ch]], add=True)
        return 0

    lax.fori_loop(0, NBLK, _block, 0)

    plsc.subcore_barrier()
    pltpu.sync_copy(acc.at[pl.ds(s * ROWS_PT, ROWS_PT)],
                    out.at[pl.ds(coff + s * ROWS_PT, ROWS_PT)])


@jax.jit
def _sc_hop(h2, src, dst2d):
    mesh = plsc.VectorSubcoreMesh(core_axis_name="c", subcore_axis_name="s")
    f = pl.kernel(
        _sc_hop_body,
        out_type=jax.ShapeDtypeStruct((2 * NP, HW), jnp.float32),
        mesh=mesh,
        scratch_types=[
            pltpu.VMEM_SHARED((NP, HW), jnp.float32),
            pltpu.VMEM((128, HW), jnp.float32),
            pltpu.VMEM((BLK,), jnp.int32),
            pltpu.VMEM((NCHB, CH), jnp.int32),
            pltpu.VMEM((NBUF, CH, HW), jnp.float32),
        ] + [pltpu.SemaphoreType.DMA] * NBUF,
        compiler_params=pltpu.CompilerParams(use_tc_tiling_on_sc=False),
    )
    return f(h2, src, dst2d)


E2 = 819200
EPAD = E2 - E
A1_BLK = 5120
A1_EPT = E2 // 32


def _sc_att_body(asd, src, dst, ef, tab, srcb, dstb, isb, asb, adb, eb0, eb1, eb2, eb3):
    c = lax.axis_index("c")
    s = lax.axis_index("s")
    t = s * 2 + c
    pltpu.sync_copy(asd.at[pl.ds(s * 25000, 25000)], tab.at[pl.ds(s * 25000, 25000)])
    plsc.subcore_barrier()
    ebase = t * A1_EPT
    ebs = [eb0, eb1, eb2, eb3]

    def _blk(blk, _):
        b0 = ebase + blk * A1_BLK
        pltpu.sync_copy(src.at[pl.ds(b0, A1_BLK)], srcb)
        pltpu.sync_copy(dst.at[pl.ds(b0, A1_BLK)], dstb)

        def _chunk(off, w):
            nw = w // 16
            for h in range(4):
                def _i1(i, _):
                    isb[pl.ds(i * 16, 16)] = srcb[pl.ds(off + i * 16, 16)] + h * N
                    return 0
                lax.fori_loop(0, nw, _i1, 0)
                pltpu.sync_copy(tab.at[isb.at[pl.ds(0, w)]], asb.at[pl.ds(0, w)])

                def _i2(i, _):
                    isb[pl.ds(i * 16, 16)] = dstb[pl.ds(off + i * 16, 16)] + (4 + h) * N
                    return 0
                lax.fori_loop(0, nw, _i2, 0)
                pltpu.sync_copy(tab.at[isb.at[pl.ds(0, w)]], adb.at[pl.ds(0, w)])

                def _e(i, _):
                    a = asb[pl.ds(i * 16, 16)] + adb[pl.ds(i * 16, 16)]
                    a = jnp.maximum(a, a * 0.2)
                    ebs[h][pl.ds(off + i * 16, 16)] = jnp.exp(a)
                    return 0
                lax.fori_loop(0, nw, _e, 0)

        def _ch(k, _):
            _chunk(k * 128, 128)
            return 0
        lax.fori_loop(0, A1_BLK // 128, _ch, 0)

        for h in range(4):
            pltpu.sync_copy(ebs[h], ef.at[pl.ds(h * E2 + b0, A1_BLK)])
        return 0
    lax.fori_loop(0, A1_EPT // A1_BLK, _blk, 0)


def _sc_att(asd, src, dst):
    mesh = plsc.VectorSubcoreMesh(core_axis_name="c", subcore_axis_name="s")
    f = pl.kernel(
        _sc_att_body,
        out_type=jax.ShapeDtypeStruct((4 * E2,), jnp.float32),
        mesh=mesh,
        scratch_types=[
            pltpu.VMEM_SHARED((8 * N,), jnp.float32),
            pltpu.VMEM((A1_BLK,), jnp.int32),
            pltpu.VMEM((A1_BLK,), jnp.int32),
            pltpu.VMEM((128,), jnp.int32),
            pltpu.VMEM((128,), jnp.float32),
            pltpu.VMEM((128,), jnp.float32),
            pltpu.VMEM((A1_BLK,), jnp.float32),
            pltpu.VMEM((A1_BLK,), jnp.float32),
            pltpu.VMEM((A1_BLK,), jnp.float32),
            pltpu.VMEM((A1_BLK,), jnp.float32),
        ],
        compiler_params=pltpu.CompilerParams(use_tc_tiling_on_sc=False),
    )
    return f(asd, src, dst)


A2_ACC = 256000
A2_EPT = E2 // 32
A2_BLK = 5120


def _sc_den_body(ef, dst, out, acc, zbuf, dstb, eb0, eb1, eb2, eb3, ones, idxb):
    c = lax.axis_index("c")
    s = lax.axis_index("s")

    def _z(i, _):
        zbuf[pl.ds(i * 16, 16)] = jnp.zeros((16,), jnp.float32)
        return 0
    lax.fori_loop(0, 128, _z, 0)

    def _o(i, _):
        ones[pl.ds(i * 16, 16)] = jnp.ones((16,), jnp.float32)
        return 0
    lax.fori_loop(0, 8, _o, 0)

    def _zcp(i, _):
        pltpu.sync_copy(zbuf, acc.at[pl.ds(s * 16000 + i * 2048, 2048)])
        return 0
    lax.fori_loop(0, 7, _zcp, 0)
    def _zcp2(i, _):
        pltpu.sync_copy(zbuf.at[pl.ds(0, 1664)], acc.at[pl.ds(s * 16000 + 14336 + i * 1664, 1664)])
        return 0
    lax.fori_loop(0, 1, _zcp2, 0)
    plsc.subcore_barrier()

    ebase = (c * 16 + s) * A2_EPT
    ebs = [eb0, eb1, eb2, eb3]

    def _blk(blk, _):
        b0 = ebase + blk * A2_BLK
        pltpu.sync_copy(dst.at[pl.ds(b0, A2_BLK)], dstb)
        for j in range(4):
            pltpu.sync_copy(ef.at[pl.ds(j * E2 + b0, A2_BLK)], ebs[j])

        def _ch(k, _):
            off = k * 128
            @pl.when(b0 + off < E)
            def _():
                def _idx(i, _2):
                    dv = dstb[pl.ds(off + i * 16, 16)]
                    for j in range(5):
                        idxb[j, pl.ds(i * 16, 16)] = dv + j * N
                    return 0
                lax.fori_loop(0, 8, _idx, 0)
                for j in range(4):
                    pltpu.sync_copy(ebs[j].at[pl.ds(off, 128)], acc.at[idxb.at[j]], add=True)
                pltpu.sync_copy(ones, acc.at[idxb.at[4]], add=True)
            return 0
        lax.fori_loop(0, A2_BLK // 128, _ch, 0)
        return 0
    lax.fori_loop(0, A2_EPT // A2_BLK, _blk, 0)

    plsc.subcore_barrier()
    pltpu.sync_copy(acc.at[pl.ds(s * 16000, 16000)],
                    out.at[pl.ds(c * A2_ACC + s * 16000, 16000)])


def _sc_den(ef, dst):
    mesh = plsc.VectorSubcoreMesh(core_axis_name="c", subcore_axis_name="s")
    f = pl.kernel(
        _sc_den_body,
        out_type=jax.ShapeDtypeStruct((2 * A2_ACC,), jnp.float32),
        mesh=mesh,
        scratch_types=[
            pltpu.VMEM_SHARED((A2_ACC,), jnp.float32),
            pltpu.VMEM((2048,), jnp.float32),
            pltpu.VMEM((A2_BLK,), jnp.int32),
            pltpu.VMEM((A2_BLK,), jnp.float32),
            pltpu.VMEM((A2_BLK,), jnp.float32),
            pltpu.VMEM((A2_BLK,), jnp.float32),
            pltpu.VMEM((A2_BLK,), jnp.float32),
            pltpu.VMEM((128,), jnp.float32),
            pltpu.VMEM((5, 128), jnp.int32),
        ],
        compiler_params=pltpu.CompilerParams(use_tc_tiling_on_sc=False),
    )
    return f(ef, dst)


def _vbcast(vec16, lane):
    idx = jnp.full((16, 1), lane, jnp.int32)
    return lax.gather(
        vec16, idx,
        lax.GatherDimensionNumbers(offset_dims=(), collapsed_slice_dims=(0,), start_index_map=(0,)),
        (1,), mode=lax.GatherScatterMode.PROMISE_IN_BOUNDS)


def _sc_gat_body(h2, ef, src, dst2d, out, acc, zbuf, idxb, dstb, rows, e0b, e1b, *gsems):
    c = lax.axis_index("c")
    s = lax.axis_index("s")

    def _z(i, _):
        zbuf[i, pl.ds(0, 16)] = jnp.zeros((16,), jnp.float32)
        zbuf[i, pl.ds(16, 16)] = jnp.zeros((16,), jnp.float32)
        return 0
    lax.fori_loop(0, 128, _z, 0)

    def _zcp(i, _):
        pltpu.sync_copy(zbuf, acc.at[pl.ds(s * ROWS_PT + i * 128, 128)])
        return 0
    lax.fori_loop(0, ROWS_PT // 128, _zcp, 0)
    plsc.subcore_barrier()

    ebase = s * EPT
    coff = c * NP

    def _gather(ch_in_blk, b):
        pltpu.async_copy(
            h2.at[idxb.at[pl.ds(ch_in_blk * CH, CH)]], rows.at[b], gsems[b])

    def _scatter(ch, b):
        pltpu.make_async_copy(h2.at[pl.ds(0, CH)], rows.at[b], gsems[b]).wait()

        def _w(g, _):
            e0v = e0b[pl.ds(ch * CH + g * 16, 16)]
            e1v = e1b[pl.ds(ch * CH + g * 16, 16)]
            for l in range(16):
                j = g * 16 + l
                e0 = _vbcast(e0v, l)
                e1 = _vbcast(e1v, l)
                rows[b, j, pl.ds(0, 16)] = rows[b, j, pl.ds(0, 16)] * e0
                rows[b, j, pl.ds(16, 16)] = rows[b, j, pl.ds(16, 16)] * e1
            return 0
        lax.fori_loop(0, CH // 16, _w, 0)
        pltpu.sync_copy(rows.at[b], acc.at[dstb.at[ch]], add=True)

    def _block(blk, _):
        b0 = ebase + blk * BLK
        pltpu.sync_copy(src.at[pl.ds(b0, BLK)], idxb)
        pltpu.sync_copy(dst2d.at[pl.ds(s * (EPT // CH) + blk * NCHB, NCHB)], dstb)
        pltpu.sync_copy(ef.at[pl.ds(2 * c * E2 + b0, BLK)], e0b)
        pltpu.sync_copy(ef.at[pl.ds((2 * c + 1) * E2 + b0, BLK)], e1b)

        def _off(i, _):
            idxb[pl.ds(i * 16, 16)] = idxb[pl.ds(i * 16, 16)] + coff
            return 0
        lax.fori_loop(0, BLK // 16, _off, 0)

        for b in range(NBUF):
            _gather(b, b)

        def _grp(g, _):
            k0 = g * NBUF
            for b in range(NBUF):
                ch = k0 + b
                _scatter(ch, b)
                _gather(ch + NBUF, b)
            return 0
        lax.fori_loop(0, (NCHB - NBUF) // NBUF, _grp, 0)

        for b in range(NBUF):
            _scatter(NCHB - NBUF + b, b)
        return 0

    lax.fori_loop(0, NBLK, _block, 0)

    plsc.subcore_barrier()
    pltpu.sync_copy(acc.at[pl.ds(s * ROWS_PT, ROWS_PT)],
                    out.at[pl.ds(coff + s * ROWS_PT, ROWS_PT)])


def _sc_gat(h2, ef, src, dst2d):
    mesh = plsc.VectorSubcoreMesh(core_axis_name="c", subcore_axis_name="s")
    f = pl.kernel(
        _sc_gat_body,
        out_type=jax.ShapeDtypeStruct((2 * NP, HW), jnp.float32),
        mesh=mesh,
        scratch_types=[
            pltpu.VMEM_SHARED((NP, HW), jnp.float32),
            pltpu.VMEM((128, HW), jnp.float32),
            pltpu.VMEM((BLK,), jnp.int32),
            pltpu.VMEM((NCHB, CH), jnp.int32),
            pltpu.VMEM((NBUF, CH, HW), jnp.float32),
            pltpu.VMEM((BLK,), jnp.float32),
            pltpu.VMEM((BLK,), jnp.float32),
        ] + [pltpu.SemaphoreType.DMA] * NBUF,
        compiler_params=pltpu.CompilerParams(use_tc_tiling_on_sc=False),
    )
    return f(h2, ef, src, dst2d)


def _split_cols(g):
    lo = jnp.pad(g[:, :HW], ((0, NP - N), (0, 0)))
    hi = jnp.pad(g[:, HW:], ((0, NP - N), (0, 0)))
    return jnp.concatenate([lo, hi], axis=0)


def _merge_cols(a2):
    return jnp.concatenate([a2[:N], a2[NP:NP + N]], axis=1)


def _graph_norm(x, batch, w, bias, ms):
    cnt = jnp.maximum(jax.ops.segment_sum(jnp.ones((x.shape[0],), x.dtype), batch, num_segments=B), 1.0)[:, None]
    mean = jax.ops.segment_sum(x, batch, num_segments=B) / cnt
    out = x - mean[batch] * ms
    var = jax.ops.segment_sum(out * out, batch, num_segments=B) / cnt
    std = jnp.sqrt(var + 1e-5)[batch]
    return w * out / std + bias


def kernel(x, edge_index, batch, W_enc, b_enc, W_gat, att_src, att_dst, b_gat, n1_w, n1_b, n1_ms, W_tag, b_tag, n2_w, n2_b, n2_ms, W_rel, b_rel, W_root, W_c1, b_c1, W_c2, b_c2):
    src = edge_index[0]
    dst2d = edge_index[1].reshape(E // CH, CH)

    x1 = jax.nn.gelu(x @ W_enc + b_enc, approximate=False)

    h = x1 @ W_gat
    hr = h.reshape(N, HEADS, DH)
    a_s = (hr * att_src[None]).sum(-1)
    a_d = (hr * att_dst[None]).sum(-1)
    asd = jnp.concatenate([a_s.T.reshape(-1), a_d.T.reshape(-1)])
    src_p = jnp.pad(src, (0, EPAD))
    dst_p = jnp.pad(edge_index[1], (0, EPAD))
    ef = _sc_att(asd, src_p, dst_p)
    part = _sc_den(ef, dst_p)
    p = part[:A2_ACC] + part[A2_ACC:]
    s4 = p[:4 * N].reshape(4, N).T
    deg = p[4 * N:5 * N]
    aggw = _merge_cols(_sc_gat(_split_cols(h), ef, src, dst2d))
    sc_self = a_s + a_d
    e_self = jnp.exp(jnp.maximum(sc_self, sc_self * 0.2))
    denom = s4 + e_self + 1e-16
    numer = aggw + (e_self[:, :, None] * hr).reshape(N, H)
    gat_out = numer / jnp.repeat(denom, DH, axis=1) + b_gat
    xg = jax.nn.relu(_graph_norm(gat_out, batch, n1_w, n1_b, n1_ms))

    dis = jnp.where(deg > 0, 1.0 / jnp.sqrt(deg), 0.0)[:, None]
    out = xg @ W_tag[0]
    h = xg
    for k in range(1, K + 1):
        agg = _merge_cols(_sc_hop(_split_cols(h * dis), src, dst2d))
        h = agg * dis
        out = out + h @ W_tag[k]
    xt = jax.nn.relu(_graph_norm(out + b_tag, batch, n2_w, n2_b, n2_ms))

    agg = _merge_cols(_sc_hop(_split_cols(xt), src, dst2d))
    xc = jax.nn.relu(agg @ W_rel + b_rel + xt @ W_root)

    cnt = jnp.maximum(jax.ops.segment_sum(jnp.ones((N,), jnp.float32), batch, num_segments=B), 1.0)[:, None]
    xm = jax.ops.segment_max(xc, batch, num_segments=B)
    xm = jnp.where(jnp.isfinite(xm), xm, 0.0)
    xM = jax.ops.segment_sum(xc, batch, num_segments=B) / cnt
    z = jnp.concatenate([xm, xM], axis=1)
    z = jax.nn.gelu(z @ W_c1 + b_c1, approximate=False)
    out = z @ W_c2 + b_c2
    return jax.nn.log_softmax(out, axis=-1)
